# baseline (speedup 1.0000x reference)
import functools

import jax
import jax.numpy as jnp
from jax import lax
from jax.experimental import pallas as pl
from jax.experimental.pallas import tpu as pltpu
from jax.experimental.pallas import tpu_sc as plsc

B = 1024
K = 20
NUM_ENT = 10000
NUM_REL = 16
S = 96
T = 32
D = S + T
R2 = 2 * NUM_REL
NTOK = 2 * B * K
BLK = 2560
EPB = BLK // K
NBLK = NTOK // BLK
TBW = 256

NW = 16
TPW = NTOK // NW
CH = 160
NCH = TPW // CH
NBUF = 3


@functools.cache
def _build_sc_gather():
    @functools.partial(
        pl.kernel,
        out_type=jax.ShapeDtypeStruct((NTOK, TBW), jnp.int32),
        mesh=plsc.VectorSubcoreMesh(core_axis_name="c", subcore_axis_name="s",
                                    num_cores=1),
        scratch_types=[
            pltpu.VMEM((TPW,), jnp.int32),
        ] + [pltpu.VMEM((CH, TBW), jnp.int32)] * NBUF
          + [pltpu.SemaphoreType.DMA] * NBUF,
    )
    def _sc_gather(table_hbm, idx_hbm, out_hbm, idx_v, *bufsem):
        bufs = bufsem[:NBUF]
        sems = bufsem[NBUF:]
        wid = lax.axis_index("s")
        base = wid * TPW
        pltpu.sync_copy(idx_hbm.at[pl.ds(base, TPW)], idx_v)
        pending = [None] * NBUF

        def start(c):
            slot = c % NBUF
            pending[slot] = pltpu.async_copy(
                table_hbm.at[idx_v.at[pl.ds(c * CH, CH)]], bufs[slot],
                sems[slot])

        for c in range(NBUF - 1):
            start(c)
        for c in range(NCH):
            if c + NBUF - 1 < NCH:
                start(c + NBUF - 1)
            slot = c % NBUF
            pending[slot].wait()
            pltpu.sync_copy(bufs[slot], out_hbm.at[pl.ds(base + c * CH, CH)])

    return _sc_gather


def _transform_body(gx_ref, gy_ref, tv_ref, rel_ref, w_ref, b_ref, pm_ref,
                    rq_ref, re_ref, out_ref, pool_ref):
    gx = gx_ref[...]
    gy = gy_ref[...]
    hmask = jnp.int32(-65536)
    g0 = lax.bitcast_convert_type(gx << 16, jnp.float32)
    fr = lax.bitcast_convert_type(gx & hmask, jnp.float32)
    ph = lax.bitcast_convert_type(gy << 16, jnp.float32)
    am = lax.bitcast_convert_type(gy & hmask, jnp.float32)

    sl = lax.broadcasted_iota(jnp.int32, (4, 128), 1)
    sc = lax.broadcasted_iota(jnp.int32, (4, 128), 0)
    sel = jnp.where((sl < S) & (sl // T == sc), 1.0, 0.0)
    tval = lax.dot_general(tv_ref[...], sel, (((1,), (0,)), ((), ())),
                           preferred_element_type=jnp.float32)

    a = fr * tval + ph
    magic = jnp.float32(12582912.0)
    nf = a * jnp.float32(0.3183098861837907) + magic
    nint = lax.bitcast_convert_type(nf, jnp.int32) - jnp.int32(0x4B400000)
    ni = nint.astype(jnp.float32)
    sgn = 1.0 - 2.0 * (nint & 1).astype(jnp.float32)
    r = a - ni * jnp.float32(3.140625)
    r = r - ni * jnp.float32(0.0009676536)
    r = r - ni * jnp.float32(5.126688e-12)
    r2 = r * r
    p = r * (jnp.float32(9.999999970017e-01)
             + r2 * (jnp.float32(-1.666665997157e-01)
                     + r2 * (jnp.float32(8.333097587152e-03)
                             + r2 * (jnp.float32(-1.981248784256e-04)
                                     + r2 * jnp.float32(2.612907779947e-06)))))
    sv = (am * (p * sgn)).astype(jnp.bfloat16)
    fl = lax.broadcasted_iota(jnp.int32, (128, 128), 0)
    fo = lax.broadcasted_iota(jnp.int32, (128, 128), 1)
    fold = jnp.where((fl < S) & (fo == S + fl % T), 1.0, 0.0).astype(jnp.bfloat16)
    temb = lax.dot_general(sv, fold, (((1,), (0,)), ((), ())),
                           preferred_element_type=jnp.float32)
    xb = (g0 + temb).astype(jnp.bfloat16)

    rel = rel_ref[...]
    ri = lax.broadcasted_iota(jnp.int32, (BLK, R2), 1)
    oh = (rel == ri).astype(jnp.float32)
    acc = lax.dot_general(oh, b_ref[...], (((1,), (0,)), ((), ())),
                          preferred_element_type=jnp.float32)
    for r_ in range(R2):
        zr = lax.dot_general(xb, w_ref[r_], (((1,), (0,)), ((), ())),
                             preferred_element_type=jnp.float32)
        acc = acc + oh[:, r_:r_ + 1] * zr
    acc = jnp.maximum(acc, 0.0)

    i = pl.program_id(0)
    pool_ref[pl.ds(i * EPB, EPB), :] = lax.dot_general(
        pm_ref[...], acc, (((1,), (0,)), ((), ())),
        preferred_element_type=jnp.float32)

    @pl.when(i == NBLK - 1)
    def _():
        h = pool_ref[0:B, :]
        t = pool_ref[B:2 * B, :]
        ridx = rq_ref[...]
        i16 = lax.broadcasted_iota(jnp.int32, (B, NUM_REL), 1)
        ohq = (ridx == i16).astype(jnp.float32)
        rr = lax.dot_general(ohq, re_ref[...], (((1,), (0,)), ((), ())),
                             preferred_element_type=jnp.float32)
        diff = h + rr - t
        s = jnp.sum(diff * diff, axis=1, keepdims=True)
        out_ref[...] = -jnp.sqrt(s + 1e-12)


_transform = pl.pallas_call(
    _transform_body,
    grid=(NBLK,),
    in_specs=[
        pl.BlockSpec((BLK, 128), lambda i: (i, 0)),
        pl.BlockSpec((BLK, 128), lambda i: (i, 1)),
        pl.BlockSpec((BLK, 4), lambda i: (i, 0)),
        pl.BlockSpec((BLK, 1), lambda i: (i, 0)),
        pl.BlockSpec((R2, D, D), lambda i: (0, 0, 0)),
        pl.BlockSpec((R2, D), lambda i: (0, 0)),
        pl.BlockSpec((EPB, BLK), lambda i: (0, 0)),
        pl.BlockSpec((B, 1), lambda i: (0, 0)),
        pl.BlockSpec((NUM_REL, D), lambda i: (0, 0)),
    ],
    out_specs=pl.BlockSpec((B, 1), lambda i: (0, 0)),
    out_shape=jax.ShapeDtypeStruct((B, 1), jnp.float32),
    scratch_shapes=[pltpu.VMEM((2 * B, D), jnp.float32)],
)


def kernel(heads, rels, tails, years, months, days, neighbor_idx, neighbor_rel,
           ny, nm, nd, ent_embs, rel_embs,
           y_freq, y_phi, y_amp, m_freq, m_phi, m_amp,
           d_freq, d_phi, d_amp, W, b):
    zpad = jnp.zeros((NUM_ENT, T), jnp.float32)
    ent_g = jnp.concatenate([ent_embs, zpad], axis=1)
    fr_g = jnp.concatenate([y_freq, m_freq, d_freq, zpad], axis=1)
    ph_g = jnp.concatenate([y_phi, m_phi, d_phi, zpad], axis=1)
    am_g = jnp.concatenate([y_amp, m_amp, d_amp, zpad], axis=1)

    def pack2(lo, hi):
        lob = lax.bitcast_convert_type(lo.astype(jnp.bfloat16),
                                       jnp.uint16).astype(jnp.uint32)
        hib = lax.bitcast_convert_type(hi.astype(jnp.bfloat16),
                                       jnp.uint16).astype(jnp.uint32)
        return lax.bitcast_convert_type(lob | (hib << 16), jnp.int32)

    table = jnp.concatenate([pack2(ent_g, fr_g), pack2(ph_g, am_g)], axis=1)
    idx = neighbor_idx.reshape(NTOK).astype(jnp.int32)
    g = _build_sc_gather()(table, idx)
    tv = jnp.concatenate(
        [ny, nm, nd, jnp.zeros_like(ny)], axis=-1).reshape(NTOK, 4)
    rel = neighbor_rel.reshape(NTOK, 1).astype(jnp.int32)
    erow = lax.broadcasted_iota(jnp.int32, (EPB, BLK), 0)
    ecol = lax.broadcasted_iota(jnp.int32, (EPB, BLK), 1)
    pmat = jnp.where(ecol // K == erow, jnp.float32(1.0 / K), 0.0)
    scores = _transform(g, g, tv, rel, W.astype(jnp.bfloat16), b, pmat,
                        rels.reshape(B, 1).astype(jnp.int32), rel_embs)
    return scores.reshape(B)


# speedup vs baseline: 1.0026x; 1.0026x over previous
import functools

import jax
import jax.numpy as jnp
from jax import lax
from jax.experimental import pallas as pl
from jax.experimental.pallas import tpu as pltpu
from jax.experimental.pallas import tpu_sc as plsc

B = 1024
K = 20
NUM_ENT = 10000
NUM_REL = 16
S = 96
T = 32
D = S + T
R2 = 2 * NUM_REL
NTOK = 2 * B * K
BLK = 2560
EPB = BLK // K
NBLK = NTOK // BLK
TBW = 256

NW = 32
TPW = NTOK // NW
CH = 160
NCH = TPW // CH
NBUF = 3


@functools.cache
def _build_sc_gather():
    @functools.partial(
        pl.kernel,
        out_type=jax.ShapeDtypeStruct((NTOK, TBW), jnp.int32),
        mesh=plsc.VectorSubcoreMesh(core_axis_name="c", subcore_axis_name="s"),
        scratch_types=[
            pltpu.VMEM((TPW,), jnp.int32),
        ] + [pltpu.VMEM((CH, TBW), jnp.int32)] * NBUF
          + [pltpu.SemaphoreType.DMA] * NBUF,
    )
    def _sc_gather(table_hbm, idx_hbm, out_hbm, idx_v, *bufsem):
        bufs = bufsem[:NBUF]
        sems = bufsem[NBUF:]
        wid = lax.axis_index("s") * 2 + lax.axis_index("c")
        base = wid * TPW
        pltpu.sync_copy(idx_hbm.at[pl.ds(base, TPW)], idx_v)
        pending = [None] * NBUF

        def start(c):
            slot = c % NBUF
            pending[slot] = pltpu.async_copy(
                table_hbm.at[idx_v.at[pl.ds(c * CH, CH)]], bufs[slot],
                sems[slot])

        for c in range(NBUF - 1):
            start(c)
        for c in range(NCH):
            if c + NBUF - 1 < NCH:
                start(c + NBUF - 1)
            slot = c % NBUF
            pending[slot].wait()
            pltpu.sync_copy(bufs[slot], out_hbm.at[pl.ds(base + c * CH, CH)])

    return _sc_gather


def _transform_body(gx_ref, gy_ref, tv_ref, rel_ref, w_ref, b_ref, pm_ref,
                    rq_ref, re_ref, out_ref, pool_ref):
    gx = gx_ref[...]
    gy = gy_ref[...]
    hmask = jnp.int32(-65536)
    g0 = lax.bitcast_convert_type(gx << 16, jnp.float32)
    fr = lax.bitcast_convert_type(gx & hmask, jnp.float32)
    ph = lax.bitcast_convert_type(gy << 16, jnp.float32)
    am = lax.bitcast_convert_type(gy & hmask, jnp.float32)

    sl = lax.broadcasted_iota(jnp.int32, (4, 128), 1)
    sc = lax.broadcasted_iota(jnp.int32, (4, 128), 0)
    sel = jnp.where((sl < S) & (sl // T == sc), 1.0, 0.0)
    tval = lax.dot_general(tv_ref[...], sel, (((1,), (0,)), ((), ())),
                           preferred_element_type=jnp.float32)

    a = fr * tval + ph
    magic = jnp.float32(12582912.0)
    nf = a * jnp.float32(0.3183098861837907) + magic
    nint = lax.bitcast_convert_type(nf, jnp.int32) - jnp.int32(0x4B400000)
    ni = nint.astype(jnp.float32)
    sgn = 1.0 - 2.0 * (nint & 1).astype(jnp.float32)
    r = a - ni * jnp.float32(3.140625)
    r = r - ni * jnp.float32(0.0009676536)
    r = r - ni * jnp.float32(5.126688e-12)
    r2 = r * r
    p = r * (jnp.float32(9.999999970017e-01)
             + r2 * (jnp.float32(-1.666665997157e-01)
                     + r2 * (jnp.float32(8.333097587152e-03)
                             + r2 * (jnp.float32(-1.981248784256e-04)
                                     + r2 * jnp.float32(2.612907779947e-06)))))
    sv = (am * (p * sgn)).astype(jnp.bfloat16)
    fl = lax.broadcasted_iota(jnp.int32, (128, 128), 0)
    fo = lax.broadcasted_iota(jnp.int32, (128, 128), 1)
    fold = jnp.where((fl < S) & (fo == S + fl % T), 1.0, 0.0).astype(jnp.bfloat16)
    temb = lax.dot_general(sv, fold, (((1,), (0,)), ((), ())),
                           preferred_element_type=jnp.float32)
    xb = (g0 + temb).astype(jnp.bfloat16)

    rel = rel_ref[...]
    ri = lax.broadcasted_iota(jnp.int32, (BLK, R2), 1)
    oh = (rel == ri).astype(jnp.float32)
    acc = lax.dot_general(oh, b_ref[...], (((1,), (0,)), ((), ())),
                          preferred_element_type=jnp.float32)
    for r_ in range(R2):
        zr = lax.dot_general(xb, w_ref[r_], (((1,), (0,)), ((), ())),
                             preferred_element_type=jnp.float32)
        acc = acc + oh[:, r_:r_ + 1] * zr
    acc = jnp.maximum(acc, 0.0)

    i = pl.program_id(0)
    pool_ref[pl.ds(i * EPB, EPB), :] = lax.dot_general(
        pm_ref[...], acc, (((1,), (0,)), ((), ())),
        preferred_element_type=jnp.float32)

    @pl.when(i == NBLK - 1)
    def _():
        h = pool_ref[0:B, :]
        t = pool_ref[B:2 * B, :]
        ridx = rq_ref[...]
        i16 = lax.broadcasted_iota(jnp.int32, (B, NUM_REL), 1)
        ohq = (ridx == i16).astype(jnp.float32)
        rr = lax.dot_general(ohq, re_ref[...], (((1,), (0,)), ((), ())),
                             preferred_element_type=jnp.float32)
        diff = h + rr - t
        s = jnp.sum(diff * diff, axis=1, keepdims=True)
        out_ref[...] = -jnp.sqrt(s + 1e-12)


_transform = pl.pallas_call(
    _transform_body,
    grid=(NBLK,),
    in_specs=[
        pl.BlockSpec((BLK, 128), lambda i: (i, 0)),
        pl.BlockSpec((BLK, 128), lambda i: (i, 1)),
        pl.BlockSpec((BLK, 4), lambda i: (i, 0)),
        pl.BlockSpec((BLK, 1), lambda i: (i, 0)),
        pl.BlockSpec((R2, D, D), lambda i: (0, 0, 0)),
        pl.BlockSpec((R2, D), lambda i: (0, 0)),
        pl.BlockSpec((EPB, BLK), lambda i: (0, 0)),
        pl.BlockSpec((B, 1), lambda i: (0, 0)),
        pl.BlockSpec((NUM_REL, D), lambda i: (0, 0)),
    ],
    out_specs=pl.BlockSpec((B, 1), lambda i: (0, 0)),
    out_shape=jax.ShapeDtypeStruct((B, 1), jnp.float32),
    scratch_shapes=[pltpu.VMEM((2 * B, D), jnp.float32)],
)


def kernel(heads, rels, tails, years, months, days, neighbor_idx, neighbor_rel,
           ny, nm, nd, ent_embs, rel_embs,
           y_freq, y_phi, y_amp, m_freq, m_phi, m_amp,
           d_freq, d_phi, d_amp, W, b):
    zpad = jnp.zeros((NUM_ENT, T), jnp.float32)
    ent_g = jnp.concatenate([ent_embs, zpad], axis=1)
    fr_g = jnp.concatenate([y_freq, m_freq, d_freq, zpad], axis=1)
    ph_g = jnp.concatenate([y_phi, m_phi, d_phi, zpad], axis=1)
    am_g = jnp.concatenate([y_amp, m_amp, d_amp, zpad], axis=1)

    def pack2(lo, hi):
        lob = lax.bitcast_convert_type(lo.astype(jnp.bfloat16),
                                       jnp.uint16).astype(jnp.uint32)
        hib = lax.bitcast_convert_type(hi.astype(jnp.bfloat16),
                                       jnp.uint16).astype(jnp.uint32)
        return lax.bitcast_convert_type(lob | (hib << 16), jnp.int32)

    table = jnp.concatenate([pack2(ent_g, fr_g), pack2(ph_g, am_g)], axis=1)
    idx = neighbor_idx.reshape(NTOK).astype(jnp.int32)
    g = _build_sc_gather()(table, idx)
    tv = jnp.concatenate(
        [ny, nm, nd, jnp.zeros_like(ny)], axis=-1).reshape(NTOK, 4)
    rel = neighbor_rel.reshape(NTOK, 1).astype(jnp.int32)
    erow = lax.broadcasted_iota(jnp.int32, (EPB, BLK), 0)
    ecol = lax.broadcasted_iota(jnp.int32, (EPB, BLK), 1)
    pmat = jnp.where(ecol // K == erow, jnp.float32(1.0 / K), 0.0)
    scores = _transform(g, g, tv, rel, W.astype(jnp.bfloat16), b, pmat,
                        rels.reshape(B, 1).astype(jnp.int32), rel_embs)
    return scores.reshape(B)
